# baseline (speedup 1.0000x reference)
import functools

import jax
import jax.numpy as jnp
from jax import lax
from jax.experimental import pallas as pl
from jax.experimental.pallas import tpu as pltpu
from jax.experimental.pallas import tpu_sc as plsc

N_NODES = 10000
N_EDGES = 320000
IN_DIM = 128
HID_DIM = 64
OUT_DIM = 32

NPAD = 10240
EPAD = 327680
NW = 32
CHUNK = 128
CPT = EPAD // NW // CHUNK
KGRP = 4
NGRP = CPT // KGRP
ROWS_PER_TILE = NPAD // 16

_mesh = plsc.VectorSubcoreMesh(core_axis_name="c", subcore_axis_name="s")
_sc_params = pltpu.CompilerParams(
    needs_layout_passes=False, use_tc_tiling_on_sc=False)


@functools.partial(
    pl.kernel,
    mesh=_mesh,
    out_type=jax.ShapeDtypeStruct((NW, NPAD), jnp.float32),
    compiler_params=_sc_params,
    scratch_types=[
        pltpu.VMEM((CPT, CHUNK), jnp.int32),
        pltpu.VMEM((NPAD,), jnp.float32),
    ],
)
def _sc_hist(dst_hbm, out_hbm, dst_v, hist_v):
    c = lax.axis_index("c")
    s = lax.axis_index("s")
    wid = c * 16 + s
    pltpu.sync_copy(dst_hbm.at[wid], dst_v)

    zeros = jnp.zeros((16,), jnp.float32)

    def zbody(i, carry):
        hist_v[pl.ds(i * 16, 16)] = zeros
        return carry

    lax.fori_loop(0, NPAD // 16, zbody, 0)

    ones = jnp.ones((16,), jnp.float32)

    def body(j, carry):
        for k in range(CHUNK // 16):
            idx = dst_v[j, pl.ds(k * 16, 16)]
            plsc.addupdate_scatter(hist_v, [idx], ones)
        return carry

    lax.fori_loop(0, CPT, body, 0)
    pltpu.sync_copy(hist_v, out_hbm.at[wid])


def _make_prop(dim):
    @functools.partial(
        pl.kernel,
        mesh=_mesh,
        out_type=jax.ShapeDtypeStruct((2, NPAD, dim), jnp.float32),
        compiler_params=_sc_params,
        scratch_types=[
            pltpu.VMEM((CPT, CHUNK), jnp.int32),
            pltpu.VMEM((CPT, CHUNK), jnp.int32),
            pltpu.VMEM((2 * KGRP, CHUNK, dim), jnp.float32),
            pltpu.VMEM_SHARED((NPAD, dim), jnp.float32),
            pltpu.SemaphoreType.DMA,
            (pltpu.SemaphoreType.DMA, pltpu.SemaphoreType.DMA),
            (pltpu.SemaphoreType.DMA, pltpu.SemaphoreType.DMA),
        ],
    )
    def _prop(src_hbm, dst_hbm, hs_hbm, zeros_hbm, out_hbm,
              src_v, dst_v, rows_v, acc_sh, sem, gsem, ssem):
        c = lax.axis_index("c")
        s = lax.axis_index("s")
        wid = c * 16 + s
        row0 = s * ROWS_PER_TILE

        pltpu.async_copy(zeros_hbm.at[pl.ds(row0, ROWS_PER_TILE)],
                         acc_sh.at[pl.ds(row0, ROWS_PER_TILE)], sem)
        pltpu.async_copy(src_hbm.at[wid], src_v, sem)
        cp = pltpu.make_async_copy(dst_hbm.at[wid], dst_v, sem)
        cp.start()
        pltpu.make_async_copy(zeros_hbm.at[pl.ds(row0, ROWS_PER_TILE)],
                              acc_sh.at[pl.ds(row0, ROWS_PER_TILE)], sem).wait()
        pltpu.make_async_copy(src_hbm.at[wid], src_v, sem).wait()
        cp.wait()
        plsc.subcore_barrier()

        def _gather(j, buf):
            pltpu.async_copy(hs_hbm.at[src_v.at[j]], rows_v.at[buf],
                             gsem[buf // KGRP])

        def _gather_wait(j, buf):
            pltpu.make_async_copy(hs_hbm.at[src_v.at[j]], rows_v.at[buf],
                                  gsem[buf // KGRP]).wait()

        def _scatter(j, buf):
            pltpu.async_copy(rows_v.at[buf], acc_sh.at[dst_v.at[j]],
                             ssem[buf // KGRP], add=True)

        def _scatter_wait(j, buf):
            pltpu.make_async_copy(rows_v.at[buf], acc_sh.at[dst_v.at[j]],
                                  ssem[buf // KGRP]).wait()

        for b in range(KGRP):
            _gather(b, b)

        def body(m, carry):
            base = m * 2 * KGRP

            @pl.when(m >= 1)
            def _():
                for b in range(KGRP):
                    _scatter_wait(base - KGRP + b, KGRP + b)

            for b in range(KGRP):
                _gather(base + KGRP + b, KGRP + b)
            for b in range(KGRP):
                _gather_wait(base + b, b)
            for b in range(KGRP):
                _scatter(base + b, b)

            for b in range(KGRP):
                _scatter_wait(base + b, b)

            @pl.when(m + 1 < NGRP // 2)
            def _():
                for b in range(KGRP):
                    _gather(base + 2 * KGRP + b, b)

            for b in range(KGRP):
                _gather_wait(base + KGRP + b, KGRP + b)
            for b in range(KGRP):
                _scatter(base + KGRP + b, KGRP + b)
            return carry

        lax.fori_loop(0, NGRP // 2, body, 0)
        for b in range(KGRP):
            _scatter_wait((NGRP - 1) * KGRP + b, KGRP + b)
        plsc.subcore_barrier()
        pltpu.sync_copy(acc_sh.at[pl.ds(row0, ROWS_PER_TILE)],
                        out_hbm.at[c, pl.ds(row0, ROWS_PER_TILE)])

    return _prop


_prop64 = _make_prop(HID_DIM)
_prop32 = _make_prop(OUT_DIM)


def _tc_prep_body(hists_ref, x_ref, w1_ref, hs_ref, dinv_ref):
    deg = jnp.sum(hists_ref[...], axis=0) + 1.0
    rows = lax.broadcasted_iota(jnp.int32, (NPAD, 1), 0)
    dinv = jnp.where(rows < N_NODES, lax.rsqrt(deg)[:, None], 0.0)
    h = jnp.dot(x_ref[...], w1_ref[...], preferred_element_type=jnp.float32)
    hs_ref[...] = h * dinv
    dinv_ref[...] = dinv


_tc_prep = pl.pallas_call(
    _tc_prep_body,
    out_shape=(
        jax.ShapeDtypeStruct((NPAD, HID_DIM), jnp.float32),
        jax.ShapeDtypeStruct((NPAD, 1), jnp.float32),
    ),
)


def _tc_combine1_body(p_ref, hs_ref, dinv_ref, b1_ref, w2_ref, hs2_ref):
    t = (p_ref[0] + p_ref[1] + hs_ref[...]) * dinv_ref[...] + b1_ref[...]
    h = jnp.maximum(t, 0.0)
    hs2_ref[...] = jnp.dot(
        h, w2_ref[...], preferred_element_type=jnp.float32) * dinv_ref[...]


_tc_combine1 = pl.pallas_call(
    _tc_combine1_body,
    out_shape=jax.ShapeDtypeStruct((NPAD, OUT_DIM), jnp.float32),
)


def _tc_final_body(p_ref, hs2_ref, dinv_ref, b2_ref, wa_ref, ba_ref, out_ref):
    t = (p_ref[0] + p_ref[1] + hs2_ref[...]) * dinv_ref[...] + b2_ref[...]
    h = jnp.maximum(t, 0.0)
    logit = jnp.sum(h * wa_ref[...], axis=-1, keepdims=True) + ba_ref[...]
    out_ref[...] = h * jax.nn.sigmoid(logit)


_tc_final = pl.pallas_call(
    _tc_final_body,
    out_shape=jax.ShapeDtypeStruct((NPAD, OUT_DIM), jnp.float32),
)


def kernel(x, edge_index, W1, b1, W2, b2, Wa, ba):
    ei = edge_index.astype(jnp.int32)
    pad = jnp.full((EPAD - N_EDGES,), N_NODES, jnp.int32)
    src = jnp.concatenate([ei[0], pad]).reshape(NW, CPT, CHUNK)
    dst = jnp.concatenate([ei[1], pad]).reshape(NW, CPT, CHUNK)

    hists = _sc_hist(dst)

    x_pad = jnp.pad(x, ((0, NPAD - N_NODES), (0, 0)))
    hs1, dinv = _tc_prep(hists, x_pad, W1)

    p1 = _prop64(src, dst, hs1, jnp.zeros((NPAD, HID_DIM), jnp.float32))
    hs2 = _tc_combine1(p1, hs1, dinv, b1.reshape(1, HID_DIM), W2)

    p2 = _prop32(src, dst, hs2, jnp.zeros((NPAD, OUT_DIM), jnp.float32))
    out = _tc_final(p2, hs2, dinv, b2.reshape(1, OUT_DIM),
                    Wa.reshape(1, OUT_DIM), ba.reshape(1, 1))
    return out[:N_NODES]


# speedup vs baseline: 2.5915x; 2.5915x over previous
import functools

import jax
import jax.numpy as jnp
from jax import lax
from jax.experimental import pallas as pl
from jax.experimental.pallas import tpu as pltpu
from jax.experimental.pallas import tpu_sc as plsc

N_NODES = 10000
N_EDGES = 320000
IN_DIM = 128
HID_DIM = 64
OUT_DIM = 32

NPAD = 10240
EPAD = 327680
NW = 32
CHUNK = 128
CPT = EPAD // NW // CHUNK
KGRP = 4
NGRP = CPT // KGRP
ROWS_PER_TILE = NPAD // 16

_mesh = plsc.VectorSubcoreMesh(core_axis_name="c", subcore_axis_name="s")
_sc_params = pltpu.CompilerParams(
    needs_layout_passes=False, use_tc_tiling_on_sc=False)


@functools.partial(
    pl.kernel,
    mesh=_mesh,
    out_type=jax.ShapeDtypeStruct((NW, NPAD), jnp.float32),
    compiler_params=_sc_params,
    scratch_types=[
        pltpu.VMEM((CPT, CHUNK), jnp.int32),
        pltpu.VMEM((NPAD,), jnp.float32),
    ],
)
def _sc_hist(dst_hbm, out_hbm, dst_v, hist_v):
    c = lax.axis_index("c")
    s = lax.axis_index("s")
    wid = c * 16 + s
    pltpu.sync_copy(dst_hbm.at[wid], dst_v)

    zeros = jnp.zeros((16,), jnp.float32)

    def zbody(i, carry):
        hist_v[pl.ds(i * 16, 16)] = zeros
        return carry

    lax.fori_loop(0, NPAD // 16, zbody, 0)

    ones = jnp.ones((16,), jnp.float32)

    def body(j, carry):
        for k in range(CHUNK // 16):
            idx = dst_v[j, pl.ds(k * 16, 16)]
            plsc.addupdate_scatter(hist_v, [idx], ones)
        return carry

    lax.fori_loop(0, CPT, body, 0)
    pltpu.sync_copy(hist_v, out_hbm.at[wid])


def _make_prop(dim):
    @functools.partial(
        pl.kernel,
        mesh=_mesh,
        out_type=jax.ShapeDtypeStruct((2, NPAD, dim), jnp.float32),
        compiler_params=_sc_params,
        scratch_types=[
            pltpu.VMEM((CPT, CHUNK), jnp.int32),
            pltpu.VMEM((CPT, CHUNK), jnp.int32),
            pltpu.VMEM((2 * KGRP, CHUNK, dim), jnp.float32),
            pltpu.VMEM_SHARED((NPAD, dim), jnp.float32),
            pltpu.SemaphoreType.DMA,
            (pltpu.SemaphoreType.DMA, pltpu.SemaphoreType.DMA),
            (pltpu.SemaphoreType.DMA, pltpu.SemaphoreType.DMA),
        ],
    )
    def _prop(src_hbm, dst_hbm, hs_hbm, zeros_hbm, out_hbm,
              src_v, dst_v, rows_v, acc_sh, sem, gsem, ssem):
        c = lax.axis_index("c")
        s = lax.axis_index("s")
        wid = c * 16 + s
        row0 = s * ROWS_PER_TILE

        pltpu.async_copy(zeros_hbm.at[pl.ds(row0, ROWS_PER_TILE)],
                         acc_sh.at[pl.ds(row0, ROWS_PER_TILE)], sem)
        pltpu.async_copy(src_hbm.at[wid], src_v, sem)
        cp = pltpu.make_async_copy(dst_hbm.at[wid], dst_v, sem)
        cp.start()
        pltpu.make_async_copy(zeros_hbm.at[pl.ds(row0, ROWS_PER_TILE)],
                              acc_sh.at[pl.ds(row0, ROWS_PER_TILE)], sem).wait()
        pltpu.make_async_copy(src_hbm.at[wid], src_v, sem).wait()
        cp.wait()
        plsc.subcore_barrier()

        def _gather(j, buf):
            pltpu.async_copy(hs_hbm.at[src_v.at[j]], rows_v.at[buf],
                             gsem[buf // KGRP])

        def _gather_wait(j, buf):
            pltpu.make_async_copy(hs_hbm.at[src_v.at[j]], rows_v.at[buf],
                                  gsem[buf // KGRP]).wait()

        def _scatter(j, buf):
            pltpu.async_copy(rows_v.at[buf], acc_sh.at[dst_v.at[j]],
                             ssem[buf // KGRP], add=True)

        def _scatter_wait(j, buf):
            pltpu.make_async_copy(rows_v.at[buf], acc_sh.at[dst_v.at[j]],
                                  ssem[buf // KGRP]).wait()

        for b in range(KGRP):
            _gather(b, b)

        def body(m, carry):
            base = m * 2 * KGRP

            @pl.when(m >= 1)
            def _():
                for b in range(KGRP):
                    _scatter_wait(base - KGRP + b, KGRP + b)

            for b in range(KGRP):
                _gather(base + KGRP + b, KGRP + b)
            for b in range(KGRP):
                _gather_wait(base + b, b)
            for b in range(KGRP):
                _scatter(base + b, b)

            for b in range(KGRP):
                _scatter_wait(base + b, b)

            @pl.when(m + 1 < NGRP // 2)
            def _():
                for b in range(KGRP):
                    _gather(base + 2 * KGRP + b, b)

            for b in range(KGRP):
                _gather_wait(base + KGRP + b, KGRP + b)
            for b in range(KGRP):
                _scatter(base + KGRP + b, KGRP + b)
            return carry

        lax.fori_loop(0, NGRP // 2, body, 0)
        for b in range(KGRP):
            _scatter_wait((NGRP - 1) * KGRP + b, KGRP + b)
        plsc.subcore_barrier()
        pltpu.sync_copy(acc_sh.at[pl.ds(row0, ROWS_PER_TILE)],
                        out_hbm.at[c, pl.ds(row0, ROWS_PER_TILE)])

    return _prop


_prop64 = _make_prop(HID_DIM)
_prop32 = _make_prop(OUT_DIM)


def _tc_prep_body(hists_ref, x_ref, w1_ref, hs_ref, dinv_ref):
    deg = jnp.sum(hists_ref[...], axis=0) + 1.0
    rows = lax.broadcasted_iota(jnp.int32, (NPAD, 1), 0)
    dinv = jnp.where(rows < N_NODES, lax.rsqrt(deg)[:, None], 0.0)
    h = jnp.dot(x_ref[...], w1_ref[...], preferred_element_type=jnp.float32)
    hs_ref[...] = h * dinv
    dinv_ref[...] = dinv


_tc_prep = pl.pallas_call(
    _tc_prep_body,
    out_shape=(
        jax.ShapeDtypeStruct((NPAD, HID_DIM), jnp.float32),
        jax.ShapeDtypeStruct((NPAD, 1), jnp.float32),
    ),
)


def _tc_combine1_body(p_ref, hs_ref, dinv_ref, b1_ref, w2_ref, hs2_ref):
    t = (p_ref[0] + p_ref[1] + hs_ref[...]) * dinv_ref[...] + b1_ref[...]
    h = jnp.maximum(t, 0.0)
    hs2_ref[...] = jnp.dot(
        h, w2_ref[...], preferred_element_type=jnp.float32) * dinv_ref[...]


_tc_combine1 = pl.pallas_call(
    _tc_combine1_body,
    out_shape=jax.ShapeDtypeStruct((NPAD, OUT_DIM), jnp.float32),
)


def _tc_final_body(p_ref, hs2_ref, dinv_ref, b2_ref, wa_ref, ba_ref, out_ref):
    t = (p_ref[0] + p_ref[1] + hs2_ref[...]) * dinv_ref[...] + b2_ref[...]
    h = jnp.maximum(t, 0.0)
    logit = jnp.sum(h * wa_ref[...], axis=-1, keepdims=True) + ba_ref[...]
    out_ref[...] = h * jax.nn.sigmoid(logit)


_tc_final = pl.pallas_call(
    _tc_final_body,
    out_shape=jax.ShapeDtypeStruct((NPAD, OUT_DIM), jnp.float32),
)


def kernel(x, edge_index, W1, b1, W2, b2, Wa, ba):
    ei = edge_index.astype(jnp.int32)
    pad = N_NODES + jnp.arange(EPAD - N_EDGES, dtype=jnp.int32) % (
        NPAD - N_NODES)
    src = jnp.concatenate([ei[0], pad]).reshape(NW, CPT, CHUNK)
    dst = jnp.concatenate([ei[1], pad]).reshape(NW, CPT, CHUNK)

    hists = _sc_hist(dst)

    x_pad = jnp.pad(x, ((0, NPAD - N_NODES), (0, 0)))
    hs1, dinv = _tc_prep(hists, x_pad, W1)

    p1 = _prop64(src, dst, hs1, jnp.zeros((NPAD, HID_DIM), jnp.float32))
    hs2 = _tc_combine1(p1, hs1, dinv, b1.reshape(1, HID_DIM), W2)

    p2 = _prop32(src, dst, hs2, jnp.zeros((NPAD, OUT_DIM), jnp.float32))
    out = _tc_final(p2, hs2, dinv, b2.reshape(1, OUT_DIM),
                    Wa.reshape(1, OUT_DIM), ba.reshape(1, 1))
    return out[:N_NODES]
